# baseline (speedup 1.0000x reference)
import functools

import jax
import jax.numpy as jnp
from jax import lax
from jax.experimental import pallas as pl
from jax.experimental.pallas import tpu as pltpu
from jax.experimental.pallas import tpu_sc as plsc

_N, _E, _F, _H, _R, _L = 10000, 320000, 128, 64, 8, 10
_RH = _R * _H

_NC, _NS = 2, 16
_NW = _NC * _NS
_CH = 128
_NCHUNK = _E // _CH
_PER_W = _NCHUNK // _NW
_EXTRA = _NCHUNK - _PER_W * _NW
_RPS = 624
_RPS_LAST = _N - _RPS * (_NS - 1)


_NBUF = 6

_SCALE = 1024.0


def _sc_body(p_hbm, gidx_hbm, dst_hbm, out_hbm, gi_v, di_v, rows_v,
             agg_sh, gsem, ssem):
    ci = lax.axis_index("c")
    si = lax.axis_index("s")
    wid = si * _NC + ci
    rows_a = rows_v.at[0]

    def zbody(t, c):
        i = t // (_H // 32)
        j = t % (_H // 32)
        rows_a[i, pl.ds(j * 32, 32)] = jnp.zeros((32,), jnp.int16)
        return c
    lax.fori_loop(0, _CH * (_H // 32), zbody, 0)
    base = pl.multiple_of(si * _RPS, 8)

    def zspread(k, c):
        pltpu.sync_copy(rows_a, agg_sh.at[pl.ds(base + k * _CH, _CH)])
        return c
    lax.fori_loop(0, _RPS // _CH, zspread, 0)

    @pl.when(si < _NS - 1)
    def _():
        pltpu.sync_copy(rows_a.at[pl.ds(0, _RPS % _CH)],
                        agg_sh.at[pl.ds(base + (_RPS // _CH) * _CH, _RPS % _CH)])

    @pl.when(si == _NS - 1)
    def _():
        pltpu.sync_copy(rows_a, agg_sh.at[pl.ds(base + (_RPS // _CH) * _CH, _CH)])

    pltpu.sync_copy(gidx_hbm.at[pl.ds(wid * _PER_W, _PER_W)],
                    gi_v.at[pl.ds(0, _PER_W)])
    pltpu.sync_copy(dst_hbm.at[pl.ds(wid * _PER_W, _PER_W)],
                    di_v.at[pl.ds(0, _PER_W)])

    @pl.when(wid < _EXTRA)
    def _():
        pltpu.sync_copy(gidx_hbm.at[pl.ds(_NW * _PER_W + wid, 1)],
                        gi_v.at[pl.ds(_PER_W, 1)])
        pltpu.sync_copy(dst_hbm.at[pl.ds(_NW * _PER_W + wid, 1)],
                        di_v.at[pl.ds(_PER_W, 1)])

    for b in range(_NBUF):
        pltpu.async_copy(p_hbm.at[gi_v.at[b]], rows_v.at[b], gsem.at[b])
    plsc.subcore_barrier()

    def body(t, carry):
        c0 = _NBUF * t
        for b in range(_NBUF):
            pltpu.make_async_copy(p_hbm.at[gi_v.at[c0 + b]],
                                  rows_v.at[b], gsem.at[b]).wait()
            pltpu.async_copy(rows_v.at[b], agg_sh.at[di_v.at[c0 + b]],
                             ssem.at[b], add=True)
        for b in range(_NBUF):
            pltpu.make_async_copy(rows_v.at[b], agg_sh.at[di_v.at[c0 + b]],
                                  ssem.at[b]).wait()

            @pl.when(t < _PER_W // _NBUF - 1)
            def _():
                pltpu.async_copy(p_hbm.at[gi_v.at[c0 + _NBUF + b]],
                                 rows_v.at[b], gsem.at[b])
        return carry
    lax.fori_loop(0, _PER_W // _NBUF, body, 0)

    @pl.when(wid < _EXTRA)
    def _():
        pltpu.async_copy(p_hbm.at[gi_v.at[_PER_W]], rows_a, gsem.at[0]).wait()
        pltpu.sync_copy(rows_a, agg_sh.at[di_v.at[_PER_W]], add=True)

    plsc.subcore_barrier()

    @pl.when(si < _NS - 1)
    def _():
        pltpu.sync_copy(agg_sh.at[pl.ds(base, _RPS)],
                        out_hbm.at[ci, pl.ds(base, _RPS)])

    @pl.when(si == _NS - 1)
    def _():
        pltpu.sync_copy(agg_sh.at[pl.ds(base, _RPS_LAST)],
                        out_hbm.at[ci, pl.ds(base, _RPS_LAST)])


def _sc_agg(p3, gidx2, dst2):
    mesh = plsc.VectorSubcoreMesh(core_axis_name="c", subcore_axis_name="s")
    return pl.kernel(
        _sc_body,
        out_type=jax.ShapeDtypeStruct((_NC, _N, _H), jnp.int16),
        mesh=mesh,
        scratch_types=[
            pltpu.VMEM((_PER_W + 1, _CH), jnp.int32),
            pltpu.VMEM((_PER_W + 1, _CH), jnp.int32),
            pltpu.VMEM((_NBUF, _CH, _H), jnp.int16),
            pltpu.VMEM_SHARED((_N, _H), jnp.int16),
            pltpu.SemaphoreType.DMA((_NBUF,)),
            pltpu.SemaphoreType.DMA((_NBUF,)),
        ],
        compiler_params=pltpu.CompilerParams(use_tc_tiling_on_sc=False),
    )(p3.reshape(_N * _R, _H), gidx2, dst2)


_BLK = 2000


def _write_p3(p_ref, h, wcat_ref):
    for g in range(_R // 2):
        p_ref[g] = jnp.round(
            jnp.dot(h, wcat_ref[:, g * 2 * _H:(g + 1) * 2 * _H],
                    preferred_element_type=jnp.float32) * _SCALE
        ).astype(jnp.int16)


def _init_body(x_ref, win_ref, bin_ref, wcat_ref, h_ref, p_ref):
    h = jnp.tanh(jnp.dot(x_ref[...], win_ref[...],
                         preferred_element_type=jnp.float32) + bin_ref[...])
    h_ref[...] = h
    _write_p3(p_ref, h, wcat_ref)


def _init_tc(x, Win, binr, Wcat0):
    return pl.pallas_call(
        _init_body,
        grid=(_N // _BLK,),
        in_specs=[
            pl.BlockSpec((_BLK, _F), lambda i: (i, 0)),
            pl.BlockSpec((_F, _H), lambda i: (0, 0)),
            pl.BlockSpec((1, _H), lambda i: (0, 0)),
            pl.BlockSpec((_H, _RH), lambda i: (0, 0)),
        ],
        out_specs=[
            pl.BlockSpec((_BLK, _H), lambda i: (i, 0)),
            pl.BlockSpec((_R // 2, _BLK, 2 * _H), lambda i: (0, i, 0)),
        ],
        out_shape=[
            jax.ShapeDtypeStruct((_N, _H), jnp.float32),
            jax.ShapeDtypeStruct((_R // 2, _N, 2 * _H), jnp.int16),
        ],
    )(x, Win, binr, Wcat0)


def _update_body(h_ref, agg_ref, wself_ref, brel_ref, w1h_ref, w1m_ref, b1_ref,
                 w2h_ref, w2m_ref, b2_ref, wcat_ref, h_out_ref, p_out_ref):
    h = h_ref[...]
    msg = ((agg_ref[0].astype(jnp.float32) + agg_ref[1].astype(jnp.float32))
           * jnp.float32(1.0 / _SCALE)
           + jnp.dot(h, wself_ref[...], preferred_element_type=jnp.float32)
           + brel_ref[...])
    mid = jnp.tanh(jnp.dot(h, w1h_ref[...], preferred_element_type=jnp.float32)
                   + jnp.dot(msg, w1m_ref[...], preferred_element_type=jnp.float32)
                   + b1_ref[...])
    hn = jnp.tanh(jnp.dot(h, w2h_ref[...], preferred_element_type=jnp.float32)
                  + jnp.dot(mid, w2m_ref[...], preferred_element_type=jnp.float32)
                  + b2_ref[...])
    h_out_ref[...] = hn
    _write_p3(p_out_ref, hn, wcat_ref)


def _update_tc(h, agg, Wself_l, brel_l, W1h, W1m, b1_l, W2h, W2m, b2_l, Wcat_n):
    return pl.pallas_call(
        _update_body,
        grid=(_N // _BLK,),
        in_specs=[
            pl.BlockSpec((_BLK, _H), lambda i: (i, 0)),
            pl.BlockSpec((_NC, _BLK, _H), lambda i: (0, i, 0)),
            pl.BlockSpec((_H, _H), lambda i: (0, 0)),
            pl.BlockSpec((1, _H), lambda i: (0, 0)),
            pl.BlockSpec((_H, 2 * _H), lambda i: (0, 0)),
            pl.BlockSpec((_H, 2 * _H), lambda i: (0, 0)),
            pl.BlockSpec((1, 2 * _H), lambda i: (0, 0)),
            pl.BlockSpec((_H, _H), lambda i: (0, 0)),
            pl.BlockSpec((2 * _H, _H), lambda i: (0, 0)),
            pl.BlockSpec((1, _H), lambda i: (0, 0)),
            pl.BlockSpec((_H, _RH), lambda i: (0, 0)),
        ],
        out_specs=[
            pl.BlockSpec((_BLK, _H), lambda i: (i, 0)),
            pl.BlockSpec((_R // 2, _BLK, 2 * _H), lambda i: (0, i, 0)),
        ],
        out_shape=[
            jax.ShapeDtypeStruct((_N, _H), jnp.float32),
            jax.ShapeDtypeStruct((_R // 2, _N, 2 * _H), jnp.int16),
        ],
    )(h, agg, Wself_l, brel_l, W1h, W1m, b1_l, W2h, W2m, b2_l, Wcat_n)


def kernel(x, edge_index, edge_type, Win, bin_, Wrel, Wself, brel, W1, b1, W2, b2):
    src = edge_index[0]
    dst = edge_index[1]
    gidx = ((edge_type >> 1) * jnp.int32(2 * _N) + src * jnp.int32(2)
            + (edge_type & 1))

    Wcat = jnp.transpose(Wrel, (0, 2, 1, 3)).reshape(_L, _H, _RH)
    W1h = W1[:, :_H, :]
    W1m = W1[:, _H:, :]
    W2h = W2[:, :_H, :]
    W2m = W2[:, _H:, :]
    binr = bin_.reshape(1, _H)
    brelr = brel.reshape(_L, 1, _H)
    b1r = b1.reshape(_L, 1, 2 * _H)
    b2r = b2.reshape(_L, 1, _H)

    gidx2 = gidx.reshape(_NCHUNK, _CH)
    dst2 = dst.reshape(_NCHUNK, _CH)

    h, P3 = _init_tc(x, Win, binr, Wcat[0])
    for l in range(_L):
        aggp = _sc_agg(P3, gidx2, dst2)
        h, P3 = _update_tc(h, aggp, Wself[l], brelr[l], W1h[l], W1m[l], b1r[l],
                           W2h[l], W2m[l], b2r[l], Wcat[(l + 1) % _L])
    return h


# speedup vs baseline: 1.2404x; 1.2404x over previous
import functools

import jax
import jax.numpy as jnp
from jax import lax
from jax.experimental import pallas as pl
from jax.experimental.pallas import tpu as pltpu
from jax.experimental.pallas import tpu_sc as plsc

_N, _E, _F, _H, _R, _L = 10000, 320000, 128, 64, 8, 10
_RH = _R * _H

_NC, _NS = 2, 16
_NW = _NC * _NS
_CH = 128
_NCHUNK = _E // _CH
_PER_W = _NCHUNK // _NW
_EXTRA = _NCHUNK - _PER_W * _NW
_RPS = 624
_RPS_LAST = _N - _RPS * (_NS - 1)


_NBUF = 6


def _sc_body(p_hbm, gidx_hbm, dst_hbm, out_hbm, gi_v, di_v, rows_v,
             agg_sh, gsem, ssem):
    ci = lax.axis_index("c")
    si = lax.axis_index("s")
    wid = si * _NC + ci
    rows_a = rows_v.at[0]

    pltpu.async_copy(gidx_hbm.at[pl.ds(wid * _PER_W, _PER_W)],
                     gi_v.at[pl.ds(0, _PER_W)], ssem.at[0])
    pltpu.async_copy(dst_hbm.at[pl.ds(wid * _PER_W, _PER_W)],
                     di_v.at[pl.ds(0, _PER_W)], ssem.at[1])

    @pl.when(wid < _EXTRA)
    def _():
        pltpu.async_copy(gidx_hbm.at[pl.ds(_NW * _PER_W + wid, 1)],
                         gi_v.at[pl.ds(_PER_W, 1)], ssem.at[2])
        pltpu.async_copy(dst_hbm.at[pl.ds(_NW * _PER_W + wid, 1)],
                         di_v.at[pl.ds(_PER_W, 1)], ssem.at[3])

    def zbody(t, c):
        i = t // (_H // 16)
        j = t % (_H // 16)
        rows_a[i, pl.ds(j * 16, 16)] = jnp.zeros((16,), jnp.float32)
        return c
    lax.fori_loop(0, _CH * (_H // 16), zbody, 0)
    base = pl.multiple_of(si * _RPS, 8)

    def zspread(k, c):
        pltpu.sync_copy(rows_a, agg_sh.at[pl.ds(base + k * _CH, _CH)])
        return c
    lax.fori_loop(0, _RPS // _CH, zspread, 0)

    @pl.when(si < _NS - 1)
    def _():
        pltpu.sync_copy(rows_a.at[pl.ds(0, _RPS % _CH)],
                        agg_sh.at[pl.ds(base + (_RPS // _CH) * _CH, _RPS % _CH)])

    @pl.when(si == _NS - 1)
    def _():
        pltpu.sync_copy(rows_a, agg_sh.at[pl.ds(base + (_RPS // _CH) * _CH, _CH)])

    pltpu.make_async_copy(gidx_hbm.at[pl.ds(wid * _PER_W, _PER_W)],
                          gi_v.at[pl.ds(0, _PER_W)], ssem.at[0]).wait()
    pltpu.make_async_copy(dst_hbm.at[pl.ds(wid * _PER_W, _PER_W)],
                          di_v.at[pl.ds(0, _PER_W)], ssem.at[1]).wait()

    @pl.when(wid < _EXTRA)
    def _():
        pltpu.make_async_copy(gidx_hbm.at[pl.ds(_NW * _PER_W + wid, 1)],
                              gi_v.at[pl.ds(_PER_W, 1)], ssem.at[2]).wait()
        pltpu.make_async_copy(dst_hbm.at[pl.ds(_NW * _PER_W + wid, 1)],
                              di_v.at[pl.ds(_PER_W, 1)], ssem.at[3]).wait()

    for b in range(_NBUF):
        pltpu.async_copy(p_hbm.at[gi_v.at[b]], rows_v.at[b], gsem.at[b])
    plsc.subcore_barrier()

    def body(t, carry):
        c0 = _NBUF * t
        for b in range(_NBUF):
            pltpu.make_async_copy(p_hbm.at[gi_v.at[c0 + b]],
                                  rows_v.at[b], gsem.at[b]).wait()
            pltpu.async_copy(rows_v.at[b], agg_sh.at[di_v.at[c0 + b]],
                             ssem.at[b], add=True)
        for b in range(_NBUF):
            pltpu.make_async_copy(rows_v.at[b], agg_sh.at[di_v.at[c0 + b]],
                                  ssem.at[b]).wait()

            @pl.when(t < _PER_W // _NBUF - 1)
            def _():
                pltpu.async_copy(p_hbm.at[gi_v.at[c0 + _NBUF + b]],
                                 rows_v.at[b], gsem.at[b])
        return carry
    lax.fori_loop(0, _PER_W // _NBUF, body, 0)

    @pl.when(wid < _EXTRA)
    def _():
        pltpu.async_copy(p_hbm.at[gi_v.at[_PER_W]], rows_a, gsem.at[0]).wait()
        pltpu.sync_copy(rows_a, agg_sh.at[di_v.at[_PER_W]], add=True)

    plsc.subcore_barrier()

    @pl.when(si < _NS - 1)
    def _():
        pltpu.sync_copy(agg_sh.at[pl.ds(base, _RPS)],
                        out_hbm.at[ci, pl.ds(base, _RPS)])

    @pl.when(si == _NS - 1)
    def _():
        pltpu.sync_copy(agg_sh.at[pl.ds(base, _RPS_LAST)],
                        out_hbm.at[ci, pl.ds(base, _RPS_LAST)])


def _sc_agg(p3, gidx2, dst2):
    mesh = plsc.VectorSubcoreMesh(core_axis_name="c", subcore_axis_name="s")
    return pl.kernel(
        _sc_body,
        out_type=jax.ShapeDtypeStruct((_NC, _N, _H), jnp.float32),
        mesh=mesh,
        scratch_types=[
            pltpu.VMEM((_PER_W + 1, _CH), jnp.int32),
            pltpu.VMEM((_PER_W + 1, _CH), jnp.int32),
            pltpu.VMEM((_NBUF, _CH, _H), jnp.float32),
            pltpu.VMEM_SHARED((_N, _H), jnp.float32),
            pltpu.SemaphoreType.DMA((_NBUF,)),
            pltpu.SemaphoreType.DMA((_NBUF,)),
        ],
        compiler_params=pltpu.CompilerParams(use_tc_tiling_on_sc=False),
    )(p3.reshape(_N * _R, _H), gidx2, dst2)


_BLK = 2000


def _write_p3(p_ref, h, wcat_ref):
    for g in range(_R // 2):
        p_ref[g] = jnp.dot(h, wcat_ref[:, g * 2 * _H:(g + 1) * 2 * _H],
                           preferred_element_type=jnp.float32)


def _init_body(x_ref, win_ref, bin_ref, wcat_ref, h_ref, p_ref):
    h = jnp.tanh(jnp.dot(x_ref[...], win_ref[...],
                         preferred_element_type=jnp.float32) + bin_ref[...])
    h_ref[...] = h
    _write_p3(p_ref, h, wcat_ref)


def _init_tc(x, Win, binr, Wcat0):
    return pl.pallas_call(
        _init_body,
        grid=(_N // _BLK,),
        in_specs=[
            pl.BlockSpec((_BLK, _F), lambda i: (i, 0)),
            pl.BlockSpec((_F, _H), lambda i: (0, 0)),
            pl.BlockSpec((1, _H), lambda i: (0, 0)),
            pl.BlockSpec((_H, _RH), lambda i: (0, 0)),
        ],
        out_specs=[
            pl.BlockSpec((_BLK, _H), lambda i: (i, 0)),
            pl.BlockSpec((_R // 2, _BLK, 2 * _H), lambda i: (0, i, 0)),
        ],
        out_shape=[
            jax.ShapeDtypeStruct((_N, _H), jnp.float32),
            jax.ShapeDtypeStruct((_R // 2, _N, 2 * _H), jnp.float32),
        ],
    )(x, Win, binr, Wcat0)


def _update_body(h_ref, agg_ref, wself_ref, brel_ref, w1h_ref, w1m_ref, b1_ref,
                 w2h_ref, w2m_ref, b2_ref, wcat_ref, h_out_ref, p_out_ref):
    h = h_ref[...]
    msg = (agg_ref[0] + agg_ref[1]
           + jnp.dot(h, wself_ref[...], preferred_element_type=jnp.float32)
           + brel_ref[...])
    mid = jnp.tanh(jnp.dot(h, w1h_ref[...], preferred_element_type=jnp.float32)
                   + jnp.dot(msg, w1m_ref[...], preferred_element_type=jnp.float32)
                   + b1_ref[...])
    hn = jnp.tanh(jnp.dot(h, w2h_ref[...], preferred_element_type=jnp.float32)
                  + jnp.dot(mid, w2m_ref[...], preferred_element_type=jnp.float32)
                  + b2_ref[...])
    h_out_ref[...] = hn
    _write_p3(p_out_ref, hn, wcat_ref)


def _update_tc(h, agg, Wself_l, brel_l, W1h, W1m, b1_l, W2h, W2m, b2_l, Wcat_n):
    return pl.pallas_call(
        _update_body,
        grid=(_N // _BLK,),
        in_specs=[
            pl.BlockSpec((_BLK, _H), lambda i: (i, 0)),
            pl.BlockSpec((_NC, _BLK, _H), lambda i: (0, i, 0)),
            pl.BlockSpec((_H, _H), lambda i: (0, 0)),
            pl.BlockSpec((1, _H), lambda i: (0, 0)),
            pl.BlockSpec((_H, 2 * _H), lambda i: (0, 0)),
            pl.BlockSpec((_H, 2 * _H), lambda i: (0, 0)),
            pl.BlockSpec((1, 2 * _H), lambda i: (0, 0)),
            pl.BlockSpec((_H, _H), lambda i: (0, 0)),
            pl.BlockSpec((2 * _H, _H), lambda i: (0, 0)),
            pl.BlockSpec((1, _H), lambda i: (0, 0)),
            pl.BlockSpec((_H, _RH), lambda i: (0, 0)),
        ],
        out_specs=[
            pl.BlockSpec((_BLK, _H), lambda i: (i, 0)),
            pl.BlockSpec((_R // 2, _BLK, 2 * _H), lambda i: (0, i, 0)),
        ],
        out_shape=[
            jax.ShapeDtypeStruct((_N, _H), jnp.float32),
            jax.ShapeDtypeStruct((_R // 2, _N, 2 * _H), jnp.float32),
        ],
    )(h, agg, Wself_l, brel_l, W1h, W1m, b1_l, W2h, W2m, b2_l, Wcat_n)


def kernel(x, edge_index, edge_type, Win, bin_, Wrel, Wself, brel, W1, b1, W2, b2):
    src = edge_index[0]
    dst = edge_index[1]
    gidx = ((edge_type >> 1) * jnp.int32(2 * _N) + src * jnp.int32(2)
            + (edge_type & 1))

    Wcat = jnp.transpose(Wrel, (0, 2, 1, 3)).reshape(_L, _H, _RH)
    W1h = W1[:, :_H, :]
    W1m = W1[:, _H:, :]
    W2h = W2[:, :_H, :]
    W2m = W2[:, _H:, :]
    binr = bin_.reshape(1, _H)
    brelr = brel.reshape(_L, 1, _H)
    b1r = b1.reshape(_L, 1, 2 * _H)
    b2r = b2.reshape(_L, 1, _H)

    gidx2 = gidx.reshape(_NCHUNK, _CH)
    dst2 = dst.reshape(_NCHUNK, _CH)

    h, P3 = _init_tc(x, Win, binr, Wcat[0])
    for l in range(_L):
        aggp = _sc_agg(P3, gidx2, dst2)
        h, P3 = _update_tc(h, aggp, Wself[l], brelr[l], W1h[l], W1m[l], b1r[l],
                           W2h[l], W2m[l], b2r[l], Wcat[(l + 1) % _L])
    return h


# speedup vs baseline: 1.2609x; 1.0165x over previous
import functools

import jax
import jax.numpy as jnp
from jax import lax
from jax.experimental import pallas as pl
from jax.experimental.pallas import tpu as pltpu
from jax.experimental.pallas import tpu_sc as plsc

_N, _E, _F, _H, _R, _L = 10000, 320000, 128, 64, 8, 10
_RH = _R * _H

_NC, _NS = 2, 16
_NW = _NC * _NS
_CH = 128
_NCHUNK = _E // _CH
_PER_W = _NCHUNK // _NW
_EXTRA = _NCHUNK - _PER_W * _NW
_RPS = 624
_RPS_LAST = _N - _RPS * (_NS - 1)


_NBUF = 6


def _sc_body(p_hbm, gidx_hbm, dst_hbm, out_hbm, gi_v, di_v, rows_v,
             agg_sh, gsem, ssem):
    ci = lax.axis_index("c")
    si = lax.axis_index("s")
    wid = si * _NC + ci
    rows_a = rows_v.at[0]

    pltpu.async_copy(gidx_hbm.at[pl.ds(wid * _PER_W, _PER_W)],
                     gi_v.at[pl.ds(0, _PER_W)], ssem.at[0])
    pltpu.async_copy(dst_hbm.at[pl.ds(wid * _PER_W, _PER_W)],
                     di_v.at[pl.ds(0, _PER_W)], ssem.at[1])

    @pl.when(wid < _EXTRA)
    def _():
        pltpu.async_copy(gidx_hbm.at[pl.ds(_NW * _PER_W + wid, 1)],
                         gi_v.at[pl.ds(_PER_W, 1)], ssem.at[2])
        pltpu.async_copy(dst_hbm.at[pl.ds(_NW * _PER_W + wid, 1)],
                         di_v.at[pl.ds(_PER_W, 1)], ssem.at[3])

    def zbody(i, c):
        for j in range(_H // 16):
            rows_a[i, pl.ds(j * 16, 16)] = jnp.zeros((16,), jnp.float32)
        return c
    lax.fori_loop(0, _CH, zbody, 0)
    base = pl.multiple_of(si * _RPS, 8)

    def zspread(k, c):
        pltpu.sync_copy(rows_a, agg_sh.at[pl.ds(base + k * _CH, _CH)])
        return c
    lax.fori_loop(0, _RPS // _CH, zspread, 0)

    @pl.when(si < _NS - 1)
    def _():
        pltpu.sync_copy(rows_a.at[pl.ds(0, _RPS % _CH)],
                        agg_sh.at[pl.ds(base + (_RPS // _CH) * _CH, _RPS % _CH)])

    @pl.when(si == _NS - 1)
    def _():
        pltpu.sync_copy(rows_a, agg_sh.at[pl.ds(base + (_RPS // _CH) * _CH, _CH)])

    pltpu.make_async_copy(gidx_hbm.at[pl.ds(wid * _PER_W, _PER_W)],
                          gi_v.at[pl.ds(0, _PER_W)], ssem.at[0]).wait()
    pltpu.make_async_copy(dst_hbm.at[pl.ds(wid * _PER_W, _PER_W)],
                          di_v.at[pl.ds(0, _PER_W)], ssem.at[1]).wait()

    @pl.when(wid < _EXTRA)
    def _():
        pltpu.make_async_copy(gidx_hbm.at[pl.ds(_NW * _PER_W + wid, 1)],
                              gi_v.at[pl.ds(_PER_W, 1)], ssem.at[2]).wait()
        pltpu.make_async_copy(dst_hbm.at[pl.ds(_NW * _PER_W + wid, 1)],
                              di_v.at[pl.ds(_PER_W, 1)], ssem.at[3]).wait()

    for b in range(_NBUF):
        pltpu.async_copy(p_hbm.at[gi_v.at[b]], rows_v.at[b], gsem.at[b])
    plsc.subcore_barrier()

    def body(t, carry):
        c0 = _NBUF * t
        for b in range(_NBUF):
            pltpu.make_async_copy(p_hbm.at[gi_v.at[c0 + b]],
                                  rows_v.at[b], gsem.at[b]).wait()
            pltpu.async_copy(rows_v.at[b], agg_sh.at[di_v.at[c0 + b]],
                             ssem.at[b], add=True)
        for b in range(_NBUF):
            pltpu.make_async_copy(rows_v.at[b], agg_sh.at[di_v.at[c0 + b]],
                                  ssem.at[b]).wait()

            @pl.when(t < _PER_W // _NBUF - 1)
            def _():
                pltpu.async_copy(p_hbm.at[gi_v.at[c0 + _NBUF + b]],
                                 rows_v.at[b], gsem.at[b])
        return carry
    lax.fori_loop(0, _PER_W // _NBUF, body, 0)

    @pl.when(wid < _EXTRA)
    def _():
        pltpu.async_copy(p_hbm.at[gi_v.at[_PER_W]], rows_a, gsem.at[0]).wait()
        pltpu.sync_copy(rows_a, agg_sh.at[di_v.at[_PER_W]], add=True)

    plsc.subcore_barrier()

    @pl.when(si < _NS - 1)
    def _():
        pltpu.sync_copy(agg_sh.at[pl.ds(base, _RPS)],
                        out_hbm.at[ci, pl.ds(base, _RPS)])

    @pl.when(si == _NS - 1)
    def _():
        pltpu.sync_copy(agg_sh.at[pl.ds(base, _RPS_LAST)],
                        out_hbm.at[ci, pl.ds(base, _RPS_LAST)])


def _sc_agg(p3, gidx2, dst2):
    mesh = plsc.VectorSubcoreMesh(core_axis_name="c", subcore_axis_name="s")
    return pl.kernel(
        _sc_body,
        out_type=jax.ShapeDtypeStruct((_NC, _N, _H), jnp.float32),
        mesh=mesh,
        scratch_types=[
            pltpu.VMEM((_PER_W + 1, _CH), jnp.int32),
            pltpu.VMEM((_PER_W + 1, _CH), jnp.int32),
            pltpu.VMEM((_NBUF, _CH, _H), jnp.float32),
            pltpu.VMEM_SHARED((_N, _H), jnp.float32),
            pltpu.SemaphoreType.DMA((_NBUF,)),
            pltpu.SemaphoreType.DMA((_NBUF,)),
        ],
        compiler_params=pltpu.CompilerParams(use_tc_tiling_on_sc=False),
    )(p3.reshape(_N * _R, _H), gidx2, dst2)


_BLK = 2000


def _write_p3(p_ref, h, wcat_ref):
    for g in range(_R // 2):
        p_ref[g] = jnp.dot(h, wcat_ref[:, g * 2 * _H:(g + 1) * 2 * _H],
                           preferred_element_type=jnp.float32)


def _init_body(x_ref, win_ref, bin_ref, wcat_ref, h_ref, p_ref):
    h = jnp.tanh(jnp.dot(x_ref[...], win_ref[...],
                         preferred_element_type=jnp.float32) + bin_ref[...])
    h_ref[...] = h
    _write_p3(p_ref, h, wcat_ref)


def _init_tc(x, Win, binr, Wcat0):
    return pl.pallas_call(
        _init_body,
        grid=(_N // _BLK,),
        in_specs=[
            pl.BlockSpec((_BLK, _F), lambda i: (i, 0)),
            pl.BlockSpec((_F, _H), lambda i: (0, 0)),
            pl.BlockSpec((1, _H), lambda i: (0, 0)),
            pl.BlockSpec((_H, _RH), lambda i: (0, 0)),
        ],
        out_specs=[
            pl.BlockSpec((_BLK, _H), lambda i: (i, 0)),
            pl.BlockSpec((_R // 2, _BLK, 2 * _H), lambda i: (0, i, 0)),
        ],
        out_shape=[
            jax.ShapeDtypeStruct((_N, _H), jnp.float32),
            jax.ShapeDtypeStruct((_R // 2, _N, 2 * _H), jnp.float32),
        ],
    )(x, Win, binr, Wcat0)


def _update_body(h_ref, agg_ref, wself_ref, brel_ref, w1h_ref, w1m_ref, b1_ref,
                 w2h_ref, w2m_ref, b2_ref, wcat_ref, h_out_ref, p_out_ref):
    h = h_ref[...]
    msg = (agg_ref[0] + agg_ref[1]
           + jnp.dot(h, wself_ref[...], preferred_element_type=jnp.float32)
           + brel_ref[...])
    mid = jnp.tanh(jnp.dot(h, w1h_ref[...], preferred_element_type=jnp.float32)
                   + jnp.dot(msg, w1m_ref[...], preferred_element_type=jnp.float32)
                   + b1_ref[...])
    hn = jnp.tanh(jnp.dot(h, w2h_ref[...], preferred_element_type=jnp.float32)
                  + jnp.dot(mid, w2m_ref[...], preferred_element_type=jnp.float32)
                  + b2_ref[...])
    h_out_ref[...] = hn
    _write_p3(p_out_ref, hn, wcat_ref)


def _update_tc(h, agg, Wself_l, brel_l, W1h, W1m, b1_l, W2h, W2m, b2_l, Wcat_n):
    return pl.pallas_call(
        _update_body,
        grid=(_N // _BLK,),
        in_specs=[
            pl.BlockSpec((_BLK, _H), lambda i: (i, 0)),
            pl.BlockSpec((_NC, _BLK, _H), lambda i: (0, i, 0)),
            pl.BlockSpec((_H, _H), lambda i: (0, 0)),
            pl.BlockSpec((1, _H), lambda i: (0, 0)),
            pl.BlockSpec((_H, 2 * _H), lambda i: (0, 0)),
            pl.BlockSpec((_H, 2 * _H), lambda i: (0, 0)),
            pl.BlockSpec((1, 2 * _H), lambda i: (0, 0)),
            pl.BlockSpec((_H, _H), lambda i: (0, 0)),
            pl.BlockSpec((2 * _H, _H), lambda i: (0, 0)),
            pl.BlockSpec((1, _H), lambda i: (0, 0)),
            pl.BlockSpec((_H, _RH), lambda i: (0, 0)),
        ],
        out_specs=[
            pl.BlockSpec((_BLK, _H), lambda i: (i, 0)),
            pl.BlockSpec((_R // 2, _BLK, 2 * _H), lambda i: (0, i, 0)),
        ],
        out_shape=[
            jax.ShapeDtypeStruct((_N, _H), jnp.float32),
            jax.ShapeDtypeStruct((_R // 2, _N, 2 * _H), jnp.float32),
        ],
    )(h, agg, Wself_l, brel_l, W1h, W1m, b1_l, W2h, W2m, b2_l, Wcat_n)


def kernel(x, edge_index, edge_type, Win, bin_, Wrel, Wself, brel, W1, b1, W2, b2):
    src = edge_index[0]
    dst = edge_index[1]
    gidx = ((edge_type >> 1) * jnp.int32(2 * _N) + src * jnp.int32(2)
            + (edge_type & 1))

    Wcat = jnp.transpose(Wrel, (0, 2, 1, 3)).reshape(_L, _H, _RH)
    W1h = W1[:, :_H, :]
    W1m = W1[:, _H:, :]
    W2h = W2[:, :_H, :]
    W2m = W2[:, _H:, :]
    binr = bin_.reshape(1, _H)
    brelr = brel.reshape(_L, 1, _H)
    b1r = b1.reshape(_L, 1, 2 * _H)
    b2r = b2.reshape(_L, 1, _H)

    gidx2 = gidx.reshape(_NCHUNK, _CH)
    dst2 = dst.reshape(_NCHUNK, _CH)

    h, P3 = _init_tc(x, Win, binr, Wcat[0])
    for l in range(_L):
        aggp = _sc_agg(P3, gidx2, dst2)
        h, P3 = _update_tc(h, aggp, Wself[l], brelr[l], W1h[l], W1m[l], b1r[l],
                           W2h[l], W2m[l], b2r[l], Wcat[(l + 1) % _L])
    return h


# speedup vs baseline: 1.2738x; 1.0102x over previous
import functools

import jax
import jax.numpy as jnp
from jax import lax
from jax.experimental import pallas as pl
from jax.experimental.pallas import tpu as pltpu
from jax.experimental.pallas import tpu_sc as plsc

_N, _E, _F, _H, _R, _L = 10000, 320000, 128, 64, 8, 10
_RH = _R * _H

_NC, _NS = 2, 16
_NW = _NC * _NS
_CH = 128
_NCHUNK = _E // _CH
_PER_W = _NCHUNK // _NW
_EXTRA = _NCHUNK - _PER_W * _NW
_RPS = 624
_RPS_LAST = _N - _RPS * (_NS - 1)


_NBUF = 6


def _sc_body(p_hbm, gidx_hbm, dst_hbm, out_hbm, gi_v, di_v, rows_v,
             agg_sh, gsem, ssem):
    ci = lax.axis_index("c")
    si = lax.axis_index("s")
    wid = si * _NC + ci
    rows_a = rows_v.at[0]

    pltpu.async_copy(gidx_hbm.at[pl.ds(wid * _PER_W, _PER_W)],
                     gi_v.at[pl.ds(0, _PER_W)], ssem.at[0])
    pltpu.async_copy(dst_hbm.at[pl.ds(wid * _PER_W, _PER_W)],
                     di_v.at[pl.ds(0, _PER_W)], ssem.at[1])

    @pl.when(wid < _EXTRA)
    def _():
        pltpu.async_copy(gidx_hbm.at[pl.ds(_NW * _PER_W + wid, 1)],
                         gi_v.at[pl.ds(_PER_W, 1)], ssem.at[2])
        pltpu.async_copy(dst_hbm.at[pl.ds(_NW * _PER_W + wid, 1)],
                         di_v.at[pl.ds(_PER_W, 1)], ssem.at[3])

    def zbody(i, c):
        for j in range(_H // 16):
            rows_a[i, pl.ds(j * 16, 16)] = jnp.zeros((16,), jnp.float32)
        return c
    lax.fori_loop(0, _CH, zbody, 0)
    base = pl.multiple_of(si * _RPS, 8)

    def zspread(k, c):
        pltpu.sync_copy(rows_a, agg_sh.at[pl.ds(base + k * _CH, _CH)])
        return c
    lax.fori_loop(0, _RPS // _CH, zspread, 0)

    @pl.when(si < _NS - 1)
    def _():
        pltpu.sync_copy(rows_a.at[pl.ds(0, _RPS % _CH)],
                        agg_sh.at[pl.ds(base + (_RPS // _CH) * _CH, _RPS % _CH)])

    @pl.when(si == _NS - 1)
    def _():
        pltpu.sync_copy(rows_a, agg_sh.at[pl.ds(base + (_RPS // _CH) * _CH, _CH)])

    pltpu.make_async_copy(gidx_hbm.at[pl.ds(wid * _PER_W, _PER_W)],
                          gi_v.at[pl.ds(0, _PER_W)], ssem.at[0]).wait()
    pltpu.make_async_copy(dst_hbm.at[pl.ds(wid * _PER_W, _PER_W)],
                          di_v.at[pl.ds(0, _PER_W)], ssem.at[1]).wait()

    @pl.when(wid < _EXTRA)
    def _():
        pltpu.make_async_copy(gidx_hbm.at[pl.ds(_NW * _PER_W + wid, 1)],
                              gi_v.at[pl.ds(_PER_W, 1)], ssem.at[2]).wait()
        pltpu.make_async_copy(dst_hbm.at[pl.ds(_NW * _PER_W + wid, 1)],
                              di_v.at[pl.ds(_PER_W, 1)], ssem.at[3]).wait()

    for b in range(_NBUF):
        pltpu.async_copy(p_hbm.at[gi_v.at[b]], rows_v.at[b], gsem.at[b])

    @pl.when(wid < _EXTRA)
    def _():
        pltpu.async_copy(p_hbm.at[gi_v.at[_PER_W]], rows_v.at[_NBUF],
                         gsem.at[_NBUF])

    plsc.subcore_barrier()

    def body(t, carry):
        c0 = _NBUF * t
        for b in range(_NBUF):
            pltpu.make_async_copy(p_hbm.at[gi_v.at[c0 + b]],
                                  rows_v.at[b], gsem.at[b]).wait()
            pltpu.async_copy(rows_v.at[b], agg_sh.at[di_v.at[c0 + b]],
                             ssem.at[b], add=True)
        for b in range(_NBUF):
            pltpu.make_async_copy(rows_v.at[b], agg_sh.at[di_v.at[c0 + b]],
                                  ssem.at[b]).wait()

            @pl.when(t < _PER_W // _NBUF - 1)
            def _():
                pltpu.async_copy(p_hbm.at[gi_v.at[c0 + _NBUF + b]],
                                 rows_v.at[b], gsem.at[b])
        return carry
    lax.fori_loop(0, _PER_W // _NBUF, body, 0)

    @pl.when(wid < _EXTRA)
    def _():
        pltpu.make_async_copy(p_hbm.at[gi_v.at[_PER_W]], rows_v.at[_NBUF],
                              gsem.at[_NBUF]).wait()
        pltpu.sync_copy(rows_v.at[_NBUF], agg_sh.at[di_v.at[_PER_W]], add=True)

    plsc.subcore_barrier()

    @pl.when(si < _NS - 1)
    def _():
        pltpu.sync_copy(agg_sh.at[pl.ds(base, _RPS)],
                        out_hbm.at[ci, pl.ds(base, _RPS)])

    @pl.when(si == _NS - 1)
    def _():
        pltpu.sync_copy(agg_sh.at[pl.ds(base, _RPS_LAST)],
                        out_hbm.at[ci, pl.ds(base, _RPS_LAST)])


def _sc_agg(p3, gidx2, dst2):
    mesh = plsc.VectorSubcoreMesh(core_axis_name="c", subcore_axis_name="s")
    return pl.kernel(
        _sc_body,
        out_type=jax.ShapeDtypeStruct((_NC, _N, _H), jnp.float32),
        mesh=mesh,
        scratch_types=[
            pltpu.VMEM((_PER_W + 1, _CH), jnp.int32),
            pltpu.VMEM((_PER_W + 1, _CH), jnp.int32),
            pltpu.VMEM((_NBUF + 1, _CH, _H), jnp.float32),
            pltpu.VMEM_SHARED((_N, _H), jnp.float32),
            pltpu.SemaphoreType.DMA((_NBUF + 1,)),
            pltpu.SemaphoreType.DMA((_NBUF,)),
        ],
        compiler_params=pltpu.CompilerParams(use_tc_tiling_on_sc=False),
    )(p3.reshape(_N * _R, _H), gidx2, dst2)


_BLK = 2000


def _write_p3(p_ref, h, wcat_ref):
    for g in range(_R // 2):
        p_ref[g] = jnp.dot(h, wcat_ref[:, g * 2 * _H:(g + 1) * 2 * _H],
                           preferred_element_type=jnp.float32)


def _init_body(x_ref, win_ref, bin_ref, wcat_ref, h_ref, p_ref):
    h = jnp.tanh(jnp.dot(x_ref[...], win_ref[...],
                         preferred_element_type=jnp.float32) + bin_ref[...])
    h_ref[...] = h
    _write_p3(p_ref, h, wcat_ref)


def _init_tc(x, Win, binr, Wcat0):
    return pl.pallas_call(
        _init_body,
        grid=(_N // _BLK,),
        in_specs=[
            pl.BlockSpec((_BLK, _F), lambda i: (i, 0)),
            pl.BlockSpec((_F, _H), lambda i: (0, 0)),
            pl.BlockSpec((1, _H), lambda i: (0, 0)),
            pl.BlockSpec((_H, _RH), lambda i: (0, 0)),
        ],
        out_specs=[
            pl.BlockSpec((_BLK, _H), lambda i: (i, 0)),
            pl.BlockSpec((_R // 2, _BLK, 2 * _H), lambda i: (0, i, 0)),
        ],
        out_shape=[
            jax.ShapeDtypeStruct((_N, _H), jnp.float32),
            jax.ShapeDtypeStruct((_R // 2, _N, 2 * _H), jnp.float32),
        ],
    )(x, Win, binr, Wcat0)


def _update_core(h_ref, agg_ref, wself_ref, brel_ref, w1h_ref, w1m_ref, b1_ref,
                 w2h_ref, w2m_ref, b2_ref):
    h = h_ref[...]
    msg = (agg_ref[0] + agg_ref[1]
           + jnp.dot(h, wself_ref[...], preferred_element_type=jnp.float32)
           + brel_ref[...])
    mid = jnp.tanh(jnp.dot(h, w1h_ref[...], preferred_element_type=jnp.float32)
                   + jnp.dot(msg, w1m_ref[...], preferred_element_type=jnp.float32)
                   + b1_ref[...])
    return jnp.tanh(jnp.dot(h, w2h_ref[...], preferred_element_type=jnp.float32)
                    + jnp.dot(mid, w2m_ref[...], preferred_element_type=jnp.float32)
                    + b2_ref[...])


def _update_body(h_ref, agg_ref, wself_ref, brel_ref, w1h_ref, w1m_ref, b1_ref,
                 w2h_ref, w2m_ref, b2_ref, wcat_ref, h_out_ref, p_out_ref):
    hn = _update_core(h_ref, agg_ref, wself_ref, brel_ref, w1h_ref, w1m_ref,
                      b1_ref, w2h_ref, w2m_ref, b2_ref)
    h_out_ref[...] = hn
    _write_p3(p_out_ref, hn, wcat_ref)


def _final_body(h_ref, agg_ref, wself_ref, brel_ref, w1h_ref, w1m_ref, b1_ref,
                w2h_ref, w2m_ref, b2_ref, h_out_ref):
    h_out_ref[...] = _update_core(h_ref, agg_ref, wself_ref, brel_ref, w1h_ref,
                                  w1m_ref, b1_ref, w2h_ref, w2m_ref, b2_ref)


_UPDATE_IN_SPECS = [
    pl.BlockSpec((_BLK, _H), lambda i: (i, 0)),
    pl.BlockSpec((_NC, _BLK, _H), lambda i: (0, i, 0)),
    pl.BlockSpec((_H, _H), lambda i: (0, 0)),
    pl.BlockSpec((1, _H), lambda i: (0, 0)),
    pl.BlockSpec((_H, 2 * _H), lambda i: (0, 0)),
    pl.BlockSpec((_H, 2 * _H), lambda i: (0, 0)),
    pl.BlockSpec((1, 2 * _H), lambda i: (0, 0)),
    pl.BlockSpec((_H, _H), lambda i: (0, 0)),
    pl.BlockSpec((2 * _H, _H), lambda i: (0, 0)),
    pl.BlockSpec((1, _H), lambda i: (0, 0)),
]


def _update_tc(h, agg, Wself_l, brel_l, W1h, W1m, b1_l, W2h, W2m, b2_l, Wcat_n):
    return pl.pallas_call(
        _update_body,
        grid=(_N // _BLK,),
        in_specs=_UPDATE_IN_SPECS + [pl.BlockSpec((_H, _RH), lambda i: (0, 0))],
        out_specs=[
            pl.BlockSpec((_BLK, _H), lambda i: (i, 0)),
            pl.BlockSpec((_R // 2, _BLK, 2 * _H), lambda i: (0, i, 0)),
        ],
        out_shape=[
            jax.ShapeDtypeStruct((_N, _H), jnp.float32),
            jax.ShapeDtypeStruct((_R // 2, _N, 2 * _H), jnp.float32),
        ],
    )(h, agg, Wself_l, brel_l, W1h, W1m, b1_l, W2h, W2m, b2_l, Wcat_n)


def _final_tc(h, agg, Wself_l, brel_l, W1h, W1m, b1_l, W2h, W2m, b2_l):
    return pl.pallas_call(
        _final_body,
        grid=(_N // _BLK,),
        in_specs=_UPDATE_IN_SPECS,
        out_specs=[pl.BlockSpec((_BLK, _H), lambda i: (i, 0))],
        out_shape=[jax.ShapeDtypeStruct((_N, _H), jnp.float32)],
    )(h, agg, Wself_l, brel_l, W1h, W1m, b1_l, W2h, W2m, b2_l)


def kernel(x, edge_index, edge_type, Win, bin_, Wrel, Wself, brel, W1, b1, W2, b2):
    src = edge_index[0]
    dst = edge_index[1]
    gidx = ((edge_type >> 1) * jnp.int32(2 * _N) + src * jnp.int32(2)
            + (edge_type & 1))

    Wcat = jnp.transpose(Wrel, (0, 2, 1, 3)).reshape(_L, _H, _RH)
    W1h = W1[:, :_H, :]
    W1m = W1[:, _H:, :]
    W2h = W2[:, :_H, :]
    W2m = W2[:, _H:, :]
    binr = bin_.reshape(1, _H)
    brelr = brel.reshape(_L, 1, _H)
    b1r = b1.reshape(_L, 1, 2 * _H)
    b2r = b2.reshape(_L, 1, _H)

    gidx2 = gidx.reshape(_NCHUNK, _CH)
    dst2 = dst.reshape(_NCHUNK, _CH)

    h, P3 = _init_tc(x, Win, binr, Wcat[0])
    for l in range(_L - 1):
        aggp = _sc_agg(P3, gidx2, dst2)
        h, P3 = _update_tc(h, aggp, Wself[l], brelr[l], W1h[l], W1m[l], b1r[l],
                           W2h[l], W2m[l], b2r[l], Wcat[l + 1])
    aggp = _sc_agg(P3, gidx2, dst2)
    l = _L - 1
    (h,) = _final_tc(h, aggp, Wself[l], brelr[l], W1h[l], W1m[l], b1r[l],
                     W2h[l], W2m[l], b2r[l])
    return h
